# baseline (speedup 1.0000x reference)
import jax
import jax.numpy as jnp
from jax import lax
from jax.experimental import pallas as pl
from jax.experimental.pallas import tpu as pltpu
from jax.experimental.pallas import tpu_sc as plsc

NC = 2
NS = 16
LANES = 16
NW = NC * NS

B, L, V, D = 4096, 200, 1000000, 64
DP = 128
N = B * L
N_PER_W = N // NW
CHUNK = 128
NBUF = 4
N_CHUNKS = N_PER_W // CHUNK
LOOKAHEAD = 3
STEADY = (N_CHUNKS - 1 - LOOKAHEAD) // NBUF


def _body(x_hbm, tok_hbm, pos_hbm, out_hbm, *refs):
  idx_all = refs[0]
  tok = refs[1:1 + NBUF]
  pos_l = refs[1 + NBUF]
  gsem = refs[2 + NBUF:2 + 2 * NBUF]
  wsem = refs[2 + 2 * NBUF:2 + 3 * NBUF]
  psem = refs[2 + 3 * NBUF:2 + 4 * NBUF]
  pos_sh = refs[2 + 4 * NBUF]

  wid = lax.axis_index("s") * NC + lax.axis_index("c")
  w_base = wid * N_PER_W

  pltpu.sync_copy(pos_hbm, pos_l.at[pl.ds(0, L + 1)])
  pltpu.sync_copy(x_hbm.at[pl.ds(w_base, N_PER_W)], idx_all)
  for r in range(LANES - 1):
    for j in range(D // LANES):
      s = pl.ds(j * LANES, LANES)
      pos_l[L + 1 + r, s] = pos_l[r + 1, s]

  @pl.when(lax.axis_index("s") == 0)
  def _():
    pltpu.sync_copy(pos_hbm, pos_sh.at[pl.ds(0, L + 1)])
    for r in range(LANES - 1):
      pltpu.sync_copy(pos_hbm.at[pl.ds(r + 1, 1)],
                      pos_sh.at[pl.ds(L + 1 + r, 1)])
  plsc.subcore_barrier()

  def fire_gather(c, k):
    base = w_base + c * CHUNK
    for g in range(CHUNK // LANES):
      l0 = lax.rem(base + g * LANES, L)
      pltpu.async_copy(pos_sh.at[pl.ds(l0 + 1, LANES)],
                       tok[k].at[pl.ds(g * LANES, LANES)], psem[k])
    for g in range(CHUNK // LANES):
      l0 = lax.rem(base + g * LANES, L)
      pltpu.make_async_copy(pos_sh.at[pl.ds(l0 + 1, LANES)],
                            tok[k].at[pl.ds(g * LANES, LANES)],
                            psem[k]).wait()
    pltpu.async_copy(tok_hbm.at[idx_all.at[pl.ds(c * CHUNK, CHUNK)]], tok[k],
                     gsem[k], add=True)

  def wait_gather(c, k):
    pltpu.make_async_copy(tok_hbm.at[idx_all.at[pl.ds(c * CHUNK, CHUNK)]],
                          tok[k], gsem[k]).wait()

  def wait_writeback(c, k):
    pltpu.make_async_copy(tok[k], out_hbm.at[pl.ds(w_base + c * CHUNK, CHUNK)],
                          wsem[k]).wait()

  def combine(c, k):
    base = w_base + c * CHUNK

    def add_body(g, _):
      xv = idx_all[pl.ds(c * CHUNK + g * LANES, LANES)]
      l0 = lax.rem(base + g * LANES, L)
      @pl.when(jnp.any(xv == 0))
      def _():
        for kk in range(LANES):
          r = g * LANES + kk

          @pl.when(xv[kk] == 0)
          def _(r=r, kk=kk):
            for j in range(D // LANES):
              s = pl.ds(j * LANES, LANES)
              tok[k][r, s] = (tok[k][r, s] + pos_l[0, s]
                              - pos_l[l0 + 1 + kk, s])
      return 0
    lax.fori_loop(0, CHUNK // LANES, add_body, 0)

    pltpu.async_copy(tok[k], out_hbm.at[pl.ds(base, CHUNK)], wsem[k])

  for c in range(LOOKAHEAD):
    fire_gather(c, c % NBUF)
  wait_gather(0, 0)
  combine(0, 0)
  fire_gather(LOOKAHEAD, LOOKAHEAD % NBUF)

  def macro_body(i, _):
    c0 = 1 + i * NBUF
    for k in range(NBUF):
      c = c0 + k
      bc = (1 + k) % NBUF
      wait_gather(c, bc)
      combine(c, bc)
      bp = (1 + k + LOOKAHEAD) % NBUF
      wait_writeback(c - 1, bp)
      fire_gather(c + LOOKAHEAD, bp)
    return 0
  lax.fori_loop(0, STEADY, macro_body, 0)

  for c in range(1 + STEADY * NBUF, N_CHUNKS):
    wait_gather(c, c % NBUF)
    combine(c, c % NBUF)

  for c in range(N_CHUNKS - NBUF, N_CHUNKS):
    wait_writeback(c, c % NBUF)


@jax.jit
def kernel(x, token_table, pos_table):
  scratch = (
      [pltpu.VMEM((N_PER_W,), jnp.int32)]
      + [pltpu.VMEM((CHUNK, DP), jnp.float32) for _ in range(NBUF)]
      + [pltpu.VMEM((L + LANES, DP), jnp.float32)]
      + [pltpu.SemaphoreType.DMA for _ in range(3 * NBUF)]
      + [pltpu.VMEM_SHARED((L + LANES, DP), jnp.float32)]
  )
  kfn = pl.kernel(
      _body,
      out_type=jax.ShapeDtypeStruct((N, DP), jnp.float32),
      mesh=plsc.VectorSubcoreMesh(core_axis_name="c", subcore_axis_name="s"),
      scratch_types=scratch,
      compiler_params=pltpu.CompilerParams(needs_layout_passes=False),
  )
  tt = jnp.pad(token_table, ((0, 0), (0, DP - D)))
  pos128 = jnp.pad(pos_table, ((0, 0), (0, DP - D)))
  out = kfn(x.reshape(N), tt, pos128)
  return out[:, :D].reshape(B, L, D)
